# Initial kernel scaffold; baseline (speedup 1.0000x reference)
import jax
import jax.numpy as jnp
from jax.experimental import pallas as pl


def kernel(x, wt_p, b_p):
    raise NotImplementedError("write your pallas kernel here")


# speedup vs baseline: 2.6537x; 2.6537x over previous
import functools

import jax
import jax.numpy as jnp
from jax.experimental import pallas as pl
from jax.experimental.pallas import tpu as pltpu


def _linear_bf16_kernel(x_ref, w_ref, b_ref, o_ref):
    xb = x_ref[...].astype(jnp.bfloat16)
    o_ref[...] = (
        jnp.dot(xb, w_ref[...], preferred_element_type=jnp.float32)
        + b_ref[...]
    ).astype(o_ref.dtype)


@functools.partial(jax.jit, static_argnames=("tm", "tn"))
def _forward(x, wt_p, b_p, *, tm, tn):
    B, K = x.shape
    _, N = wt_p.shape
    w_bf = wt_p.astype(jnp.bfloat16)
    grid = (N // tn, B // tm)
    return pl.pallas_call(
        _linear_bf16_kernel,
        out_shape=jax.ShapeDtypeStruct((B, N), x.dtype),
        grid=grid,
        in_specs=[
            pl.BlockSpec((tm, K), lambda j, i: (i, 0)),
            pl.BlockSpec((K, tn), lambda j, i: (0, j)),
            pl.BlockSpec((1, tn), lambda j, i: (0, j)),
        ],
        out_specs=pl.BlockSpec((tm, tn), lambda j, i: (i, j)),
        compiler_params=pltpu.CompilerParams(
            dimension_semantics=("parallel", "arbitrary"),
            vmem_limit_bytes=100 * 1024 * 1024,
        ),
    )(x, w_bf, b_p)


def kernel(x, wt_p, b_p):
    B, K = x.shape
    N = wt_p.shape[1]
    tm = 1024 if B % 1024 == 0 else 512
    tn = 2048 if N % 2048 == 0 and N // 2048 >= 2 else 1024
    return _forward(x, wt_p, b_p, tm=tm, tn=tn)
